# Initial kernel scaffold; baseline (speedup 1.0000x reference)
import jax
import jax.numpy as jnp
from jax.experimental import pallas as pl


def kernel(x, edge_index_0, edge_index_1, edge_index_2, edge_index_3, weight, bias, w_ih, w_hh, b_ih, b_hh):
    raise NotImplementedError("write your pallas kernel here")


# speedup vs baseline: 2.4762x; 2.4762x over previous
import functools

import jax
import jax.numpy as jnp
from jax import lax
from jax.experimental import pallas as pl
from jax.experimental.pallas import tpu as pltpu
from jax.experimental.pallas import tpu_sc as plsc

N = 10000
D = 128
T = 3
ET = 4
EDGES = 80000

NC = 2
NS = 16
CHUNK = 128
PER_CORE_RAW = ET * EDGES // NC
PER_TILE = 10240
NCHUNK = PER_TILE // CHUNK
PER_CORE = PER_TILE * NS
ACC_ROWS = 10240
ZROWS = 16
BR = 400


def _messages_body(h_ref, w_ref, b_ref, out_ref):
    out_ref[0] = (
        jnp.dot(h_ref[...], w_ref[0], preferred_element_type=jnp.float32)
        + b_ref[0, 0]
    )


def _messages(h, w, b):
    return pl.pallas_call(
        _messages_body,
        grid=(ET, N // BR),
        in_specs=[
            pl.BlockSpec((BR, D), lambda e, r: (r, 0)),
            pl.BlockSpec((1, D, D), lambda e, r: (e, 0, 0)),
            pl.BlockSpec((1, 1, D), lambda e, r: (e, 0, 0)),
        ],
        out_specs=pl.BlockSpec((1, BR, D), lambda e, r: (e, r, 0)),
        out_shape=jax.ShapeDtypeStruct((ET, N, D), jnp.float32),
    )(h, w, b)


def _gru_math(p_ref, h_ref, wih_ref, whh_ref, bih_ref, bhh_ref):
    msum = p_ref[0] + p_ref[1]
    h = h_ref[...]
    gi = jnp.dot(msum, wih_ref[...], preferred_element_type=jnp.float32) + bih_ref[0]
    gh = jnp.dot(h, whh_ref[...], preferred_element_type=jnp.float32) + bhh_ref[0]
    r = jax.nn.sigmoid(gi[:, :D] + gh[:, :D])
    z = jax.nn.sigmoid(gi[:, D:2 * D] + gh[:, D:2 * D])
    n = jnp.tanh(gi[:, 2 * D:] + r * gh[:, 2 * D:])
    return (1.0 - z) * n + z * h


def _gru_m_body(p_ref, h_ref, wih_ref, whh_ref, bih_ref, bhh_ref,
                wn_ref, bn_ref, hout_ref, mout_ref):
    hn = _gru_math(p_ref, h_ref, wih_ref, whh_ref, bih_ref, bhh_ref)
    hout_ref[...] = hn
    for e in range(ET):
        mout_ref[e] = (
            jnp.dot(hn, wn_ref[e], preferred_element_type=jnp.float32)
            + bn_ref[e]
        )


def _gru_body(p_ref, h_ref, wih_ref, whh_ref, bih_ref, bhh_ref, hout_ref):
    hout_ref[...] = _gru_math(p_ref, h_ref, wih_ref, whh_ref, bih_ref, bhh_ref)


_GRU_IN_SPECS = [
    pl.BlockSpec((2, BR, D), lambda r: (0, r, 0)),
    pl.BlockSpec((BR, D), lambda r: (r, 0)),
    pl.BlockSpec((D, 3 * D), lambda r: (0, 0)),
    pl.BlockSpec((D, 3 * D), lambda r: (0, 0)),
    pl.BlockSpec((1, 3 * D), lambda r: (0, 0)),
    pl.BlockSpec((1, 3 * D), lambda r: (0, 0)),
]


def _gru_m(parts, h, wih_t, whh_t, b_ih2, b_hh2, wn, bn):
    return pl.pallas_call(
        _gru_m_body,
        grid=(N // BR,),
        in_specs=_GRU_IN_SPECS + [
            pl.BlockSpec((ET, D, D), lambda r: (0, 0, 0)),
            pl.BlockSpec((ET, 1, D), lambda r: (0, 0, 0)),
        ],
        out_specs=[
            pl.BlockSpec((BR, D), lambda r: (r, 0)),
            pl.BlockSpec((ET, BR, D), lambda r: (0, r, 0)),
        ],
        out_shape=[
            jax.ShapeDtypeStruct((N, D), jnp.float32),
            jax.ShapeDtypeStruct((ET, N, D), jnp.float32),
        ],
    )(parts, h, wih_t, whh_t, b_ih2, b_hh2, wn, bn)


def _gru_last(parts, h, wih_t, whh_t, b_ih2, b_hh2):
    return pl.pallas_call(
        _gru_body,
        grid=(N // BR,),
        in_specs=_GRU_IN_SPECS,
        out_specs=pl.BlockSpec((BR, D), lambda r: (r, 0)),
        out_shape=jax.ShapeDtypeStruct((N, D), jnp.float32),
    )(parts, h, wih_t, whh_t, b_ih2, b_hh2)


def _sc_segment_sum(m2d, src_all, dst_all):
    mesh = plsc.VectorSubcoreMesh(
        core_axis_name="c", subcore_axis_name="s",
        num_cores=NC, num_subcores=NS,
    )

    @functools.partial(
        pl.kernel,
        out_type=jax.ShapeDtypeStruct((NC, ACC_ROWS, D), jnp.float32),
        mesh=mesh,
        scratch_types=[
            pltpu.VMEM((CHUNK,), jnp.int32),
            pltpu.VMEM((CHUNK,), jnp.int32),
            pltpu.VMEM((CHUNK, D), jnp.float32),
            pltpu.VMEM((ZROWS, D), jnp.float32),
            pltpu.VMEM_SHARED((ACC_ROWS, D), jnp.float32),
            pltpu.SemaphoreType.DMA,
        ],
    )
    def sc_kernel(m_hbm, src_hbm, dst_hbm, out_hbm,
                  sidx, didx, rows, zbuf, acc, sem):
        c = lax.axis_index("c")
        s = lax.axis_index("s")

        zero16 = jnp.zeros((16,), jnp.float32)
        for i in range(ZROWS):
            for j in range(D // 16):
                zbuf[i, pl.ds(j * 16, 16)] = zero16
        zrows_per_tile = ACC_ROWS // NS
        zbase = s * zrows_per_tile
        for i in range(zrows_per_tile // ZROWS):
            pltpu.sync_copy(zbuf, acc.at[pl.ds(zbase + i * ZROWS, ZROWS)])
        plsc.subcore_barrier()

        base = (c * NS + s) * PER_TILE

        def chunk_body(j, carry):
            off = base + j * CHUNK
            pltpu.sync_copy(src_hbm.at[pl.ds(off, CHUNK)], sidx)
            pltpu.sync_copy(dst_hbm.at[pl.ds(off, CHUNK)], didx)
            pltpu.async_copy(m_hbm.at[sidx], rows, sem).wait()
            pltpu.sync_copy(rows, acc.at[didx], add=True)
            return carry

        lax.fori_loop(0, NCHUNK, chunk_body, 0)
        plsc.subcore_barrier()

        orows = ACC_ROWS // NS
        obase = s * orows
        pltpu.sync_copy(acc.at[pl.ds(obase, orows)],
                        out_hbm.at[c, pl.ds(obase, orows)])

    return sc_kernel(m2d, src_all, dst_all)


def _pad_edges(edges):
    srcs, dsts = [], []
    for e, ei in enumerate(edges):
        dsts.append(ei[0])
        srcs.append(ei[1] + e * N)
    half = ET // NC
    src_parts, dst_parts = [], []
    pad = PER_CORE - PER_CORE_RAW
    for c in range(NC):
        s = jnp.concatenate(srcs[c * half:(c + 1) * half])
        d = jnp.concatenate(dsts[c * half:(c + 1) * half])
        src_parts.append(jnp.pad(s, (0, pad), constant_values=0))
        dst_parts.append(jnp.pad(d, (0, pad), constant_values=N))
    return jnp.concatenate(src_parts), jnp.concatenate(dst_parts)


@jax.jit
def kernel(x, edge_index_0, edge_index_1, edge_index_2, edge_index_3,
           weight, bias, w_ih, w_hh, b_ih, b_hh):
    src_all, dst_all = _pad_edges(
        [edge_index_0, edge_index_1, edge_index_2, edge_index_3])
    wih_t = w_ih.T
    whh_t = w_hh.T
    b_ih2 = b_ih.reshape(1, 3 * D)
    b_hh2 = b_hh.reshape(1, 3 * D)
    bias3 = bias.reshape(T, ET, 1, D)

    h = x
    m = _messages(x, weight[0], bias3[0])
    for t in range(T):
        parts = _sc_segment_sum(m.reshape(ET * N, D), src_all, dst_all)
        if t < T - 1:
            h, m = _gru_m(parts, h, wih_t, whh_t, b_ih2, b_hh2,
                          weight[t + 1], bias3[t + 1])
        else:
            h = _gru_last(parts, h, wih_t, whh_t, b_ih2, b_hh2)
    return h


# speedup vs baseline: 2.9292x; 1.1829x over previous
import functools

import jax
import jax.numpy as jnp
from jax import lax
from jax.experimental import pallas as pl
from jax.experimental.pallas import tpu as pltpu
from jax.experimental.pallas import tpu_sc as plsc

N = 10000
D = 128
T = 3
ET = 4
EDGES = 80000

NC = 2
NS = 16
CHUNK = 80
PER_CORE_RAW = ET * EDGES // NC
PER_TILE = 10240
NCHUNK = PER_TILE // CHUNK
PER_CORE = PER_TILE * NS
ACC_ROWS = 10240
ZROWS = 16
BR = 400


def _messages_body(h_ref, w_ref, b_ref, out_ref):
    out_ref[0] = (
        jnp.dot(h_ref[...], w_ref[0], preferred_element_type=jnp.float32)
        + b_ref[0, 0]
    )


def _messages(h, w, b):
    return pl.pallas_call(
        _messages_body,
        grid=(ET, N // BR),
        in_specs=[
            pl.BlockSpec((BR, D), lambda e, r: (r, 0)),
            pl.BlockSpec((1, D, D), lambda e, r: (e, 0, 0)),
            pl.BlockSpec((1, 1, D), lambda e, r: (e, 0, 0)),
        ],
        out_specs=pl.BlockSpec((1, BR, D), lambda e, r: (e, r, 0)),
        out_shape=jax.ShapeDtypeStruct((ET, N, D), jnp.float32),
    )(h, w, b)


def _gru_math(p_ref, h_ref, wih_ref, whh_ref, bih_ref, bhh_ref):
    msum = p_ref[0] + p_ref[1]
    h = h_ref[...]
    gi = jnp.dot(msum, wih_ref[...], preferred_element_type=jnp.float32) + bih_ref[0]
    gh = jnp.dot(h, whh_ref[...], preferred_element_type=jnp.float32) + bhh_ref[0]
    r = jax.nn.sigmoid(gi[:, :D] + gh[:, :D])
    z = jax.nn.sigmoid(gi[:, D:2 * D] + gh[:, D:2 * D])
    n = jnp.tanh(gi[:, 2 * D:] + r * gh[:, 2 * D:])
    return (1.0 - z) * n + z * h


def _gru_m_body(p_ref, h_ref, wih_ref, whh_ref, bih_ref, bhh_ref,
                wn_ref, bn_ref, hout_ref, mout_ref):
    hn = _gru_math(p_ref, h_ref, wih_ref, whh_ref, bih_ref, bhh_ref)
    hout_ref[...] = hn
    for e in range(ET):
        mout_ref[e] = (
            jnp.dot(hn, wn_ref[e], preferred_element_type=jnp.float32)
            + bn_ref[e]
        )


def _gru_body(p_ref, h_ref, wih_ref, whh_ref, bih_ref, bhh_ref, hout_ref):
    hout_ref[...] = _gru_math(p_ref, h_ref, wih_ref, whh_ref, bih_ref, bhh_ref)


_GRU_IN_SPECS = [
    pl.BlockSpec((2, BR, D), lambda r: (0, r, 0)),
    pl.BlockSpec((BR, D), lambda r: (r, 0)),
    pl.BlockSpec((D, 3 * D), lambda r: (0, 0)),
    pl.BlockSpec((D, 3 * D), lambda r: (0, 0)),
    pl.BlockSpec((1, 3 * D), lambda r: (0, 0)),
    pl.BlockSpec((1, 3 * D), lambda r: (0, 0)),
]


def _gru_m(parts, h, wih_t, whh_t, b_ih2, b_hh2, wn, bn):
    return pl.pallas_call(
        _gru_m_body,
        grid=(N // BR,),
        in_specs=_GRU_IN_SPECS + [
            pl.BlockSpec((ET, D, D), lambda r: (0, 0, 0)),
            pl.BlockSpec((ET, 1, D), lambda r: (0, 0, 0)),
        ],
        out_specs=[
            pl.BlockSpec((BR, D), lambda r: (r, 0)),
            pl.BlockSpec((ET, BR, D), lambda r: (0, r, 0)),
        ],
        out_shape=[
            jax.ShapeDtypeStruct((N, D), jnp.float32),
            jax.ShapeDtypeStruct((ET, N, D), jnp.float32),
        ],
    )(parts, h, wih_t, whh_t, b_ih2, b_hh2, wn, bn)


def _gru_last(parts, h, wih_t, whh_t, b_ih2, b_hh2):
    return pl.pallas_call(
        _gru_body,
        grid=(N // BR,),
        in_specs=_GRU_IN_SPECS,
        out_specs=pl.BlockSpec((BR, D), lambda r: (r, 0)),
        out_shape=jax.ShapeDtypeStruct((N, D), jnp.float32),
    )(parts, h, wih_t, whh_t, b_ih2, b_hh2)


def _sc_segment_sum(m2d, src_all, dst_all):
    mesh = plsc.VectorSubcoreMesh(
        core_axis_name="c", subcore_axis_name="s",
        num_cores=NC, num_subcores=NS,
    )

    @functools.partial(
        pl.kernel,
        out_type=jax.ShapeDtypeStruct((NC, ACC_ROWS, D), jnp.float32),
        mesh=mesh,
        scratch_types=[
            pltpu.VMEM((PER_TILE,), jnp.int32),
            pltpu.VMEM((NCHUNK, CHUNK), jnp.int32),
            pltpu.VMEM((CHUNK, D), jnp.float32),
            pltpu.VMEM((CHUNK, D), jnp.float32),
            pltpu.VMEM((ZROWS, D), jnp.float32),
            pltpu.VMEM_SHARED((ACC_ROWS, D), jnp.float32),
            pltpu.SemaphoreType.DMA,
            pltpu.SemaphoreType.DMA,
            pltpu.SemaphoreType.DMA,
        ],
    )
    def sc_kernel(m_hbm, src_hbm, dst_hbm, out_hbm,
                  sidx, didx, rows0, rows1, zbuf, acc, sema, semb, semz):
        c = lax.axis_index("c")
        s = lax.axis_index("s")
        tile = c * NS + s

        cp_s = pltpu.async_copy(
            src_hbm.at[pl.ds(tile * PER_TILE, PER_TILE)], sidx, sema)
        cp_d = pltpu.async_copy(
            dst_hbm.at[pl.ds(tile * NCHUNK, NCHUNK)], didx, semb)

        zero16 = jnp.zeros((16,), jnp.float32)
        for i in range(ZROWS):
            for j in range(D // 16):
                zbuf[i, pl.ds(j * 16, 16)] = zero16
        zrows_per_tile = ACC_ROWS // NS
        zbase = s * zrows_per_tile
        zcopies = [
            pltpu.async_copy(
                zbuf, acc.at[pl.ds(zbase + i * ZROWS, ZROWS)], semz)
            for i in range(zrows_per_tile // ZROWS)
        ]
        cp_s.wait()
        cp_d.wait()
        for cp in zcopies:
            cp.wait()
        plsc.subcore_barrier()

        def sl(j):
            return sidx.at[pl.ds(j * CHUNK, CHUNK)]

        pltpu.async_copy(m_hbm.at[sl(0)], rows0, sema)

        def pair_body(i, carry):
            j = 2 * i
            cpb = pltpu.async_copy(m_hbm.at[sl(j + 1)], rows1, semb)
            pltpu.make_async_copy(m_hbm.at[sl(j)], rows0, sema).wait()
            pltpu.sync_copy(rows0, acc.at[didx.at[j]], add=True)

            @pl.when(j + 2 < NCHUNK)
            def _():
                pltpu.async_copy(m_hbm.at[sl(j + 2)], rows0, sema)

            cpb.wait()
            pltpu.sync_copy(rows1, acc.at[didx.at[j + 1]], add=True)
            return carry

        lax.fori_loop(0, NCHUNK // 2, pair_body, 0)
        plsc.subcore_barrier()

        orows = ACC_ROWS // NS
        obase = s * orows
        pltpu.sync_copy(acc.at[pl.ds(obase, orows)],
                        out_hbm.at[c, pl.ds(obase, orows)])

    return sc_kernel(m2d, src_all, dst_all)


def _pad_edges(edges):
    srcs, dsts = [], []
    for e, ei in enumerate(edges):
        dsts.append(ei[0])
        srcs.append(ei[1] + e * N)
    half = ET // NC
    src_parts, dst_parts = [], []
    pad = PER_CORE - PER_CORE_RAW
    for c in range(NC):
        s = jnp.concatenate(srcs[c * half:(c + 1) * half])
        d = jnp.concatenate(dsts[c * half:(c + 1) * half])
        src_parts.append(jnp.pad(s, (0, pad), constant_values=0))
        dst_parts.append(jnp.pad(d, (0, pad), constant_values=N))
    return (jnp.concatenate(src_parts),
            jnp.concatenate(dst_parts).reshape(-1, CHUNK))


@jax.jit
def kernel(x, edge_index_0, edge_index_1, edge_index_2, edge_index_3,
           weight, bias, w_ih, w_hh, b_ih, b_hh):
    src_all, dst_all = _pad_edges(
        [edge_index_0, edge_index_1, edge_index_2, edge_index_3])
    wih_t = w_ih.T
    whh_t = w_hh.T
    b_ih2 = b_ih.reshape(1, 3 * D)
    b_hh2 = b_hh.reshape(1, 3 * D)
    bias3 = bias.reshape(T, ET, 1, D)

    h = x
    m = _messages(x, weight[0], bias3[0])
    for t in range(T):
        parts = _sc_segment_sum(m.reshape(ET * N, D), src_all, dst_all)
        if t < T - 1:
            h, m = _gru_m(parts, h, wih_t, whh_t, b_ih2, b_hh2,
                          weight[t + 1], bias3[t + 1])
        else:
            h = _gru_last(parts, h, wih_t, whh_t, b_ih2, b_hh2)
    return h
